# Initial kernel scaffold; baseline (speedup 1.0000x reference)
import jax
import jax.numpy as jnp
from jax.experimental import pallas as pl


def kernel(x, edge_index, batch, W1, a_src1, a_dst1, b1, W2, a_src2, a_dst2, b2, Wl, bl):
    raise NotImplementedError("write your pallas kernel here")


# speedup vs baseline: 12.3012x; 12.3012x over previous
import functools

import jax
import jax.numpy as jnp
from jax import lax
from jax.experimental import pallas as pl
from jax.experimental.pallas import tpu as pltpu
from jax.experimental.pallas import tpu_sc as plsc

N = 10000
E = 320000
H = 256
NSLICE = 4
FSL = H // NSLICE
NC = 2
NS = 16
NW = NC * NS
EPW_A = E // NW
EPS_B = E // NS
KCH = 400


def _mm_body(x_ref, w_ref, b_ref, o_ref, *, relu):
    o = jnp.dot(x_ref[...], w_ref[...], preferred_element_type=jnp.float32)
    o = o + b_ref[...]
    if relu:
        o = jnp.maximum(o, 0.0)
    o_ref[...] = o


def _tc_matmul(x, w, b, relu=False, bm=1000):
    m, k = x.shape
    n = w.shape[1]
    if m % bm != 0:
        bm = m
    grid = (m // bm,)
    return pl.pallas_call(
        functools.partial(_mm_body, relu=relu),
        grid=grid,
        in_specs=[
            pl.BlockSpec((bm, k), lambda i: (i, 0)),
            pl.BlockSpec((k, n), lambda i: (0, 0)),
            pl.BlockSpec((1, n), lambda i: (0, 0)),
        ],
        out_specs=pl.BlockSpec((bm, n), lambda i: (i, 0)),
        out_shape=jax.ShapeDtypeStruct((m, n), jnp.float32),
    )(x, w, b.reshape(1, n))


_MESH = plsc.VectorSubcoreMesh(core_axis_name="c", subcore_axis_name="s")


def _phase_a_body(s_hbm, t_hbm, src_hbm, dst_hbm, ex_hbm, den_hbm,
                  s_v, t_v, den_v, src_v, dst_v, ex_v):
    wid = lax.axis_index("s") * NC + lax.axis_index("c")
    base = wid * EPW_A
    pltpu.sync_copy(s_hbm, s_v)
    pltpu.sync_copy(t_hbm, t_v)
    pltpu.sync_copy(src_hbm.at[pl.ds(base, EPW_A)], src_v)
    pltpu.sync_copy(dst_hbm.at[pl.ds(base, EPW_A)], dst_v)

    zero = jnp.zeros((16,), jnp.float32)

    def zbody(i, c):
        den_v[pl.ds(i * 16, 16)] = zero
        return c

    lax.fori_loop(0, N // 16, zbody, 0)

    def ebody(i, c):
        sl = pl.ds(i * 16, 16)
        src16 = src_v[sl]
        dst16 = dst_v[sl]
        sv = plsc.load_gather(s_v, [src16])
        tv = plsc.load_gather(t_v, [dst16])
        e = sv + tv
        e = jnp.where(e > 0, e, 0.2 * e)
        ex = jnp.exp(e)
        ex_v[sl] = ex
        plsc.addupdate_scatter(den_v, [dst16], ex)
        return c

    lax.fori_loop(0, EPW_A // 16, ebody, 0)
    pltpu.sync_copy(ex_v, ex_hbm.at[pl.ds(base, EPW_A)])
    pltpu.sync_copy(den_v, den_hbm.at[wid])


_SC_PARAMS = pltpu.CompilerParams(
    needs_layout_passes=False, use_tc_tiling_on_sc=False)

_phase_a = functools.partial(
    pl.kernel,
    mesh=_MESH,
    compiler_params=_SC_PARAMS,
    out_type=[
        jax.ShapeDtypeStruct((E,), jnp.float32),
        jax.ShapeDtypeStruct((NW, N), jnp.float32),
    ],
    scratch_types=[
        pltpu.VMEM((N,), jnp.float32),
        pltpu.VMEM((N,), jnp.float32),
        pltpu.VMEM((N,), jnp.float32),
        pltpu.VMEM((EPW_A,), jnp.int32),
        pltpu.VMEM((EPW_A,), jnp.int32),
        pltpu.VMEM((EPW_A,), jnp.float32),
    ],
)(_phase_a_body)


def _phase_b_body(hflat_hbm, ex_hbm, den_hbm, src_hbm, dst_hbm, zeros_hbm,
                  out_hbm, den_v, src_c, dst_c, ax_c, off_c, rows, acc, sem):
    cid = lax.axis_index("c")
    sid = lax.axis_index("s")
    pltpu.sync_copy(den_hbm, den_v)
    ebase = sid * EPS_B

    for c in range(NC):
        for fi in range(NSLICE // NC):
            f = c * (NSLICE // NC) + fi

            @pl.when(cid == c)
            def _(f=f):
                @pl.when(sid == 0)
                def _():
                    pltpu.sync_copy(zeros_hbm, acc)

                plsc.subcore_barrier()

                def chunk(ci, carry):
                    cb = ebase + ci * KCH
                    pltpu.sync_copy(src_hbm.at[pl.ds(cb, KCH)], src_c)
                    pltpu.sync_copy(dst_hbm.at[pl.ds(cb, KCH)], dst_c)
                    pltpu.sync_copy(ex_hbm.at[pl.ds(cb, KCH)], ax_c)

                    def ab(j, cc):
                        sl = pl.ds(j * 16, 16)
                        d16 = dst_c[sl]
                        den16 = plsc.load_gather(den_v, [d16])
                        ax_c[sl] = ax_c[sl] / (den16 + 1e-16)
                        off_c[sl] = src_c[sl] + (f * N)
                        return cc

                    lax.fori_loop(0, KCH // 16, ab, 0)
                    pltpu.async_copy(hflat_hbm.at[off_c], rows, sem).wait()

                    def sb(j, cc):
                        jv = jnp.zeros((16,), jnp.int32) + j
                        a = plsc.load_gather(ax_c, [jv])
                        for q in range(FSL // 16):
                            qs = pl.ds(q * 16, 16)
                            rows[j, qs] = rows[j, qs] * a
                        return cc

                    lax.fori_loop(0, KCH, sb, 0)
                    pltpu.sync_copy(rows, acc.at[dst_c], add=True)
                    return carry

                lax.fori_loop(0, EPS_B // KCH, chunk, 0)
                plsc.subcore_barrier()

                @pl.when(sid == 0)
                def _():
                    pltpu.sync_copy(acc, out_hbm.at[f])

                plsc.subcore_barrier()


_phase_b = functools.partial(
    pl.kernel,
    mesh=_MESH,
    compiler_params=_SC_PARAMS,
    out_type=jax.ShapeDtypeStruct((NSLICE, N, FSL), jnp.float32),
    scratch_types=[
        pltpu.VMEM((N,), jnp.float32),
        pltpu.VMEM((KCH,), jnp.int32),
        pltpu.VMEM((KCH,), jnp.int32),
        pltpu.VMEM((KCH,), jnp.float32),
        pltpu.VMEM((KCH,), jnp.int32),
        pltpu.VMEM((KCH, FSL), jnp.float32),
        pltpu.VMEM_SHARED((N, FSL), jnp.float32),
        pltpu.SemaphoreType.DMA,
    ],
)(_phase_b_body)


def _gat_prop(h, src, dst, a_s, a_d, w_pad):
    st = _tc_matmul(h, w_pad, jnp.zeros((128,), jnp.float32))
    s = st[:, 0]
    t = st[:, 1]
    ex, den_parts = _phase_a(s, t, src, dst)
    den = den_parts.sum(axis=0)
    hflat = h.reshape(N, NSLICE, FSL).transpose(1, 0, 2).reshape(NSLICE * N, FSL)
    zeros64 = jnp.zeros((N, FSL), jnp.float32)
    out = _phase_b(hflat, ex, den, src, dst, zeros64)
    return out.transpose(1, 0, 2).reshape(N, H)


def _pad_st(a_s, a_d):
    w = jnp.zeros((H, 128), jnp.float32)
    return w.at[:, 0].set(a_s).at[:, 1].set(a_d)


def kernel(x, edge_index, batch, W1, a_src1, a_dst1, b1, W2, a_src2, a_dst2, b2, Wl, bl):
    src = edge_index[0]
    dst = edge_index[1]
    wst1 = _pad_st(a_src1, a_dst1)
    wst2 = _pad_st(a_src2, a_dst2)

    h = _tc_matmul(x, W1, jnp.zeros((H,), jnp.float32))
    h = _gat_prop(h, src, dst, a_src1, a_dst1, wst1)
    h = jax.nn.relu(h + b1)

    h = _tc_matmul(h, W2, jnp.zeros((H,), jnp.float32))
    for _ in range(3):
        h = _gat_prop(h, src, dst, a_src2, a_dst2, wst2)
    h = jax.nn.relu(h + b2)


    G = 64
    onehot = (batch[None, :] == jnp.arange(G, dtype=batch.dtype)[:, None]).astype(jnp.float32)
    cnt = onehot.sum(axis=1, keepdims=True)
    pmat = onehot / jnp.maximum(cnt, 1.0)
    pooled = _tc_matmul(pmat, h, jnp.zeros((H,), jnp.float32), bm=G)
    return _tc_matmul(pooled, Wl, bl, bm=G)
